# Initial kernel scaffold; baseline (speedup 1.0000x reference)
import jax
import jax.numpy as jnp
from jax.experimental import pallas as pl


def kernel(din, weight, bias):
    raise NotImplementedError("write your pallas kernel here")


# speedup vs baseline: 43.5041x; 43.5041x over previous
import jax
import jax.numpy as jnp
from jax import lax
from jax.experimental import pallas as pl
from jax.experimental.pallas import tpu as pltpu
from jax.experimental.pallas import tpu_sc as plsc

NC = 2
NS = 16
NW = NC * NS
LANES = 16


def _sc_body(f_in, f_out, din_hbm, w_hbm, b_hbm, out_hbm, x_v, o_v, w_s, b_s):
    n = din_hbm.shape[0]
    rows_per_w = n // f_in // NW
    words_per_w = rows_per_w * f_in
    wid = lax.axis_index("s") * NC + lax.axis_index("c")
    base = wid * words_per_w

    pltpu.sync_copy(din_hbm.at[pl.ds(base, words_per_w)], x_v)
    pltpu.sync_copy(w_hbm, w_s)
    pltpu.sync_copy(b_hbm, b_s)

    lanes32 = lax.iota(jnp.int32, LANES) * f_in
    bvecs = [b_s[pl.ds(h * LANES, LANES)] for h in range(f_in // LANES)]

    def group(g, carry):
        gbase = lanes32 + g * (LANES * f_in)
        acc = [jnp.zeros((LANES,), jnp.float32) for _ in range(f_out)]
        for j in range(f_in):
            xj = plsc.load_gather(x_v, [gbase + j])
            xb = xj + bvecs[j // LANES][j % LANES]
            wv = [w_s[pl.ds(j * f_out + h * LANES, LANES)]
                  for h in range(f_out // LANES)]
            for k in range(f_out):
                acc[k] = acc[k] + xb * wv[k // LANES][k % LANES]
        for k in range(f_out):
            plsc.store_scatter(o_v, [gbase + k], jnp.maximum(acc[k], 0.0))
        return carry

    lax.fori_loop(0, rows_per_w // LANES, group, 0)
    pltpu.sync_copy(o_v, out_hbm.at[pl.ds(base, words_per_w)])


@jax.jit
def kernel(din, weight, bias):
    b, f_in = din.shape
    f_out = weight.shape[1]
    words_per_w = b * f_in // NW
    mesh = plsc.VectorSubcoreMesh(
        core_axis_name="c", subcore_axis_name="s",
        num_cores=NC, num_subcores=NS)
    f = pl.kernel(
        lambda *refs: _sc_body(f_in, f_out, *refs),
        out_type=jax.ShapeDtypeStruct((b * f_out,), jnp.float32),
        mesh=mesh,
        scratch_types=[
            pltpu.VMEM((words_per_w,), jnp.float32),
            pltpu.VMEM((words_per_w,), jnp.float32),
            pltpu.VMEM((f_in * f_out,), jnp.float32),
            pltpu.VMEM((f_in,), jnp.float32),
        ],
        compiler_params=pltpu.CompilerParams(needs_layout_passes=False),
    )
    out = f(din.reshape(-1), weight.reshape(-1), bias)
    return out.reshape(b, f_out)
